# baseline (speedup 1.0000x reference)
import jax
import jax.numpy as jnp
from jax import lax
from jax.experimental import pallas as pl
from jax.experimental.pallas import tpu as pltpu
from jax.experimental.pallas import tpu_sc as plsc

N = 10000
E = 160000
D = 256
H = D // 2
NC = 2
NS = 16
CH = 128
NP = 10112
NPA = 5120

ES = E // NS
SCH = (ES + CH - 1) // CH
ACC_ROWS = NPA // NS

_mesh = plsc.VectorSubcoreMesh(core_axis_name="c", subcore_axis_name="s")


EW = E // (NC * NS)
EWP = 5120


def _deg_body(dstp_hbm, zeros_hbm, out_hbm, dst_v, cnt_v):
    c = lax.axis_index("c")
    s = lax.axis_index("s")
    pltpu.sync_copy(zeros_hbm, cnt_v)
    pltpu.sync_copy(dstp_hbm.at[c, s], dst_v)

    def body(e, carry):
        idx = dst_v[pl.ds(e * 16, 16)]
        cnts, last = plsc.scan_count(idx)
        plsc.addupdate_scatter(cnt_v, [idx], cnts, mask=last)
        return carry

    lax.fori_loop(0, EWP // 16, body, 0)
    pltpu.sync_copy(cnt_v, out_hbm.at[c, s])


_deg_call = pl.kernel(
    _deg_body,
    out_type=jax.ShapeDtypeStruct((NC, NS, NP), jnp.int32),
    mesh=_mesh,
    scratch_types=[
        pltpu.VMEM((EWP,), jnp.int32),
        pltpu.VMEM((NP,), jnp.int32),
    ],
    compiler_params=pltpu.CompilerParams(needs_layout_passes=False),
)


def _dmerge_body(cnt32_ref, o_ref):
    deg = jnp.sum(cnt32_ref[...].astype(jnp.float32), axis=0) + 1.0
    dinv = lax.rsqrt(deg)
    o_ref[...] = jnp.broadcast_to(dinv[:, None], (128, 16))


def _dmerge(cnt32):
    return pl.pallas_call(
        _dmerge_body,
        grid=(NP // 128,),
        in_specs=[pl.BlockSpec((NC * NS, 128), lambda j: (0, j))],
        out_specs=pl.BlockSpec((128, 16), lambda j: (j, 0)),
        out_shape=jax.ShapeDtypeStruct((NP, 16), jnp.float32),
    )(cnt32)


def _agg_body(hs_hbm, srcp_hbm, dstp_hbm, zeros_hbm, out_hbm,
              src_v, dst_v, buf, acc, gsem):
    c = lax.axis_index("c")
    s = lax.axis_index("s")
    pltpu.sync_copy(zeros_hbm, acc.at[pl.ds(s * ACC_ROWS, ACC_ROWS)])
    pltpu.sync_copy(srcp_hbm.at[c, s], src_v)
    pltpu.sync_copy(dstp_hbm.at[s], dst_v)
    plsc.subcore_barrier()

    def gather(j):
        return pltpu.async_copy(
            hs_hbm.at[src_v.at[pl.ds(j * CH, CH)]], buf.at[j % 2], gsem)

    pending = gather(0)
    for j in range(SCH):
        nxt = gather(j + 1) if j + 1 < SCH else None
        pending.wait()
        pltpu.sync_copy(buf.at[j % 2], acc.at[dst_v.at[j]], add=True)
        pending = nxt
    plsc.subcore_barrier()
    pltpu.sync_copy(acc.at[pl.ds(s * ACC_ROWS, ACC_ROWS)],
                    out_hbm.at[c, pl.ds(s * ACC_ROWS, ACC_ROWS)])


_agg_call = pl.kernel(
    _agg_body,
    out_type=jax.ShapeDtypeStruct((NC, NPA, H), jnp.float32),
    mesh=_mesh,
    scratch_types=[
        pltpu.VMEM((SCH * CH,), jnp.int32),
        pltpu.VMEM((SCH, CH), jnp.int32),
        pltpu.VMEM((2, CH, H), jnp.float32),
        pltpu.VMEM_SHARED((NPA, H), jnp.float32),
        pltpu.SemaphoreType.DMA,
    ],
)


_BN = 1000
_NB = N // _BN
_BNT = NP // 16
_NBT = 16


def _dinv_of(cnt_ref):
    return cnt_ref[:, 0:1]


def _rowmask(i, nrows):
    rows = i * nrows + lax.broadcasted_iota(jnp.int32, (nrows, 1), 0)
    return rows < N


def _mm_body(x_ref, w_ref, cnt_ref, o_ref):
    h = jnp.dot(x_ref[...], w_ref[...], preferred_element_type=jnp.float32)
    hs = jnp.where(_rowmask(pl.program_id(0), _BNT),
                   h * _dinv_of(cnt_ref), 0.0)
    o_ref[0] = hs[:, :H]
    o_ref[1] = hs[:, H:]


def _mm_scale(x, W, cnt):
    return pl.pallas_call(
        _mm_body,
        grid=(_NBT,),
        in_specs=[
            pl.BlockSpec((_BNT, D), lambda i: (i, 0)),
            pl.BlockSpec((D, D), lambda i: (0, 0)),
            pl.BlockSpec((_BNT, 16), lambda i: (i, 0)),
        ],
        out_specs=pl.BlockSpec((NC, _BNT, H), lambda i: (0, i, 0)),
        out_shape=jax.ShapeDtypeStruct((NC, NP, H), jnp.float32),
    )(x, W, cnt)


def _mida_body(agg_ref, tab_ref, cnt_ref, b_ref, z_ref, st_ref):
    dinv = _dinv_of(cnt_ref)
    a = jnp.concatenate([agg_ref[0], agg_ref[1]], axis=1)
    hs = jnp.concatenate([tab_ref[0], tab_ref[1]], axis=1)
    z = (a + hs) * dinv + b_ref[...]
    z_ref[...] = z

    @pl.when(pl.program_id(0) == 0)
    def _():
        st_ref[...] = jnp.zeros_like(st_ref)

    ssum = jnp.sum(z, axis=0, keepdims=True)
    ssq = jnp.sum(z * z, axis=0, keepdims=True)
    st_ref[...] += jnp.concatenate(
        [ssum, ssq, jnp.zeros((6, D), jnp.float32)], axis=0)


def _mid_a(agg, tab, cnt, b):
    return pl.pallas_call(
        _mida_body,
        grid=(_NB,),
        in_specs=[
            pl.BlockSpec((NC, _BN, H), lambda i: (0, i, 0)),
            pl.BlockSpec((NC, _BN, H), lambda i: (0, i, 0)),
            pl.BlockSpec((_BN, 16), lambda i: (i, 0)),
            pl.BlockSpec((1, D), lambda i: (0, 0)),
        ],
        out_specs=[
            pl.BlockSpec((_BN, D), lambda i: (i, 0)),
            pl.BlockSpec((8, D), lambda i: (0, 0)),
        ],
        out_shape=[
            jax.ShapeDtypeStruct((N, D), jnp.float32),
            jax.ShapeDtypeStruct((8, D), jnp.float32),
        ],
    )(agg, tab, cnt, b)


def _bn_stats(st_ref):
    mean = st_ref[0:1] * (1.0 / N)
    ex2 = st_ref[1:2] * (1.0 / N)
    var = ex2 - mean * mean
    return mean, lax.rsqrt(var + 1e-5)


def _midb_body(z_ref, st_ref, cnt_ref, g_ref, be_ref, w_ref, o_ref):
    mean, rstd = _bn_stats(st_ref)
    y = (z_ref[...] - mean) * (rstd * g_ref[...]) + be_ref[...]
    y = jnp.maximum(y, 0.0)
    h = jnp.dot(y, w_ref[...], preferred_element_type=jnp.float32)
    hs = jnp.where(_rowmask(pl.program_id(0), _BNT),
                   h * _dinv_of(cnt_ref), 0.0)
    o_ref[0] = hs[:, :H]
    o_ref[1] = hs[:, H:]


def _mid_b(z, st, cnt, gamma, beta, W):
    return pl.pallas_call(
        _midb_body,
        grid=(_NBT,),
        in_specs=[
            pl.BlockSpec((_BNT, D), lambda i: (i, 0)),
            pl.BlockSpec((8, D), lambda i: (0, 0)),
            pl.BlockSpec((_BNT, 16), lambda i: (i, 0)),
            pl.BlockSpec((1, D), lambda i: (0, 0)),
            pl.BlockSpec((1, D), lambda i: (0, 0)),
            pl.BlockSpec((D, D), lambda i: (0, 0)),
        ],
        out_specs=pl.BlockSpec((NC, _BNT, H), lambda i: (0, i, 0)),
        out_shape=jax.ShapeDtypeStruct((NC, NP, H), jnp.float32),
    )(z, st, cnt, gamma, beta, W)


def _final_body(z_ref, st_ref, g_ref, be_ref, o_ref):
    mean, rstd = _bn_stats(st_ref)
    o_ref[...] = (z_ref[...] - mean) * (rstd * g_ref[...]) + be_ref[...]


def _final(z, st, gamma, beta):
    return pl.pallas_call(
        _final_body,
        grid=(_NB,),
        in_specs=[
            pl.BlockSpec((_BN, D), lambda i: (i, 0)),
            pl.BlockSpec((8, D), lambda i: (0, 0)),
            pl.BlockSpec((1, D), lambda i: (0, 0)),
            pl.BlockSpec((1, D), lambda i: (0, 0)),
        ],
        out_specs=pl.BlockSpec((_BN, D), lambda i: (i, 0)),
        out_shape=jax.ShapeDtypeStruct((N, D), jnp.float32),
    )(z, st, gamma, beta)


def _agg_ranges(tab, srcps, dstps, zeros_a):
    flat = tab.reshape(NC * NP, H)
    o0 = _agg_call(flat, srcps[0], dstps[0], zeros_a)
    o1 = _agg_call(flat, srcps[1], dstps[1], zeros_a)
    return jnp.concatenate([o0, o1], axis=1)


def kernel(x, edge_index, W1, b1, gamma1, beta1, W2, b2, gamma2, beta2):
    src = edge_index[0]
    dst = edge_index[1]

    pads = SCH * CH - ES
    pos = jnp.arange(SCH * CH, dtype=jnp.int32)

    dst_sub = jnp.concatenate(
        [dst.reshape(NS, ES),
         jnp.full((NS, pads), N, jnp.int32)], axis=1)
    src_sub = jnp.concatenate(
        [src.reshape(NS, ES), jnp.zeros((NS, pads), jnp.int32)], axis=1)

    padw = EWP - EW
    trash_w = N + (jnp.arange(padw, dtype=jnp.int32) % (NP - N))
    dstp_deg = jnp.concatenate(
        [dst.reshape(NC * NS, EW),
         jnp.broadcast_to(trash_w, (NC * NS, padw))],
        axis=1).reshape(NC, NS, EWP).astype(jnp.int32)

    zero_rows = N + (pos % (NP - N))
    srcps, dstps = [], []
    for r in range(NC):
        in_r = ((dst_sub >= r * NPA) &
                (dst_sub < min((r + 1) * NPA, N)))
        s_r = jnp.where(in_r, src_sub, zero_rows[None, :]).astype(jnp.int32)
        d_r = jnp.where(in_r, dst_sub - r * NPA,
                        pos[None, :] % NPA).astype(jnp.int32)
        srcps.append(jnp.stack([s_r, s_r + NP]))
        dstps.append(d_r.reshape(NS, SCH, CH))

    zeros_d = jnp.zeros((NP,), jnp.int32)
    zeros_a = jnp.zeros((ACC_ROWS, H), jnp.float32)

    cnt32 = _deg_call(dstp_deg, zeros_d)
    cnt = _dmerge(cnt32.reshape(NC * NS, NP))

    tab1 = _mm_scale(x, W1, cnt)
    agg1 = _agg_ranges(tab1, srcps, dstps, zeros_a)
    z1, st1 = _mid_a(agg1, tab1, cnt, b1.reshape(1, D))
    tab2 = _mid_b(z1, st1, cnt, gamma1.reshape(1, D), beta1.reshape(1, D),
                  W2)
    agg2 = _agg_ranges(tab2, srcps, dstps, zeros_a)
    z2, st2 = _mid_a(agg2, tab2, cnt, b2.reshape(1, D))
    return _final(z2, st2, gamma2.reshape(1, D), beta2.reshape(1, D))


# speedup vs baseline: 1.0073x; 1.0073x over previous
import jax
import jax.numpy as jnp
from jax import lax
from jax.experimental import pallas as pl
from jax.experimental.pallas import tpu as pltpu
from jax.experimental.pallas import tpu_sc as plsc

N = 10000
E = 160000
D = 256
H = D // 2
NC = 2
NS = 16
CH = 128
NP = 10112
NPA = 5120

ES = E // NS
SCH = (ES + CH - 1) // CH
ACC_ROWS = NPA // NS

_mesh = plsc.VectorSubcoreMesh(core_axis_name="c", subcore_axis_name="s")


EW = E // (NC * NS)
EWP = 5120


def _deg_body(dstp_hbm, zeros_hbm, out_hbm, dst_v, cnt_v):
    c = lax.axis_index("c")
    s = lax.axis_index("s")
    pltpu.sync_copy(zeros_hbm, cnt_v)
    pltpu.sync_copy(dstp_hbm.at[c, s], dst_v)

    def body(e, carry):
        idx = dst_v[pl.ds(e * 16, 16)]
        cnts, last = plsc.scan_count(idx)
        plsc.addupdate_scatter(cnt_v, [idx], cnts, mask=last)
        return carry

    lax.fori_loop(0, EWP // 16, body, 0)
    pltpu.sync_copy(cnt_v, out_hbm.at[c, s])


_deg_call = pl.kernel(
    _deg_body,
    out_type=jax.ShapeDtypeStruct((NC, NS, NP), jnp.int32),
    mesh=_mesh,
    scratch_types=[
        pltpu.VMEM((EWP,), jnp.int32),
        pltpu.VMEM((NP,), jnp.int32),
    ],
    compiler_params=pltpu.CompilerParams(needs_layout_passes=False),
)


def _dmerge_body(cnt32_ref, o_ref):
    deg = jnp.sum(cnt32_ref[...].astype(jnp.float32), axis=0) + 1.0
    dinv = lax.rsqrt(deg)
    o_ref[...] = jnp.broadcast_to(dinv[:, None], (128, 16))


def _dmerge(cnt32):
    return pl.pallas_call(
        _dmerge_body,
        grid=(NP // 128,),
        in_specs=[pl.BlockSpec((NC * NS, 128), lambda j: (0, j))],
        out_specs=pl.BlockSpec((128, 16), lambda j: (j, 0)),
        out_shape=jax.ShapeDtypeStruct((NP, 16), jnp.float32),
    )(cnt32)


def _agg_body(hs_hbm, srcp_hbm, dstp_hbm, zeros_hbm, out_hbm,
              src_v, dst_v, buf, acc, gsem):
    c = lax.axis_index("c")
    s = lax.axis_index("s")
    pltpu.sync_copy(zeros_hbm, acc.at[pl.ds(s * ACC_ROWS, ACC_ROWS)])
    pltpu.sync_copy(srcp_hbm.at[c, s], src_v)
    pltpu.sync_copy(dstp_hbm.at[s], dst_v)
    plsc.subcore_barrier()

    def gather(j):
        return pltpu.async_copy(
            hs_hbm.at[src_v.at[pl.ds(j * CH, CH)]], buf.at[j % 3], gsem)

    gd = {0: gather(0), 1: gather(1)}
    for j in range(SCH):
        if j + 2 < SCH:
            gd[j + 2] = gather(j + 2)
        gd.pop(j).wait()
        pltpu.sync_copy(buf.at[j % 3], acc.at[dst_v.at[j]], add=True)
    plsc.subcore_barrier()
    pltpu.sync_copy(acc.at[pl.ds(s * ACC_ROWS, ACC_ROWS)],
                    out_hbm.at[c, pl.ds(s * ACC_ROWS, ACC_ROWS)])


_agg_call = pl.kernel(
    _agg_body,
    out_type=jax.ShapeDtypeStruct((NC, NPA, H), jnp.float32),
    mesh=_mesh,
    scratch_types=[
        pltpu.VMEM((SCH * CH,), jnp.int32),
        pltpu.VMEM((SCH, CH), jnp.int32),
        pltpu.VMEM((3, CH, H), jnp.float32),
        pltpu.VMEM_SHARED((NPA, H), jnp.float32),
        pltpu.SemaphoreType.DMA,
    ],
)


_BN = 1000
_NB = N // _BN
_BNT = NP // 16
_NBT = 16


def _dinv_of(cnt_ref):
    return cnt_ref[:, 0:1]


def _rowmask(i, nrows):
    rows = i * nrows + lax.broadcasted_iota(jnp.int32, (nrows, 1), 0)
    return rows < N


def _mm_body(x_ref, w_ref, cnt_ref, o_ref):
    h = jnp.dot(x_ref[...], w_ref[...], preferred_element_type=jnp.float32)
    hs = jnp.where(_rowmask(pl.program_id(0), _BNT),
                   h * _dinv_of(cnt_ref), 0.0)
    o_ref[0] = hs[:, :H]
    o_ref[1] = hs[:, H:]


def _mm_scale(x, W, cnt):
    return pl.pallas_call(
        _mm_body,
        grid=(_NBT,),
        in_specs=[
            pl.BlockSpec((_BNT, D), lambda i: (i, 0)),
            pl.BlockSpec((D, D), lambda i: (0, 0)),
            pl.BlockSpec((_BNT, 16), lambda i: (i, 0)),
        ],
        out_specs=pl.BlockSpec((NC, _BNT, H), lambda i: (0, i, 0)),
        out_shape=jax.ShapeDtypeStruct((NC, NP, H), jnp.float32),
    )(x, W, cnt)


def _mida_body(agg_ref, tab_ref, cnt_ref, b_ref, z_ref, st_ref):
    dinv = _dinv_of(cnt_ref)
    a = jnp.concatenate([agg_ref[0], agg_ref[1]], axis=1)
    hs = jnp.concatenate([tab_ref[0], tab_ref[1]], axis=1)
    z = (a + hs) * dinv + b_ref[...]
    z_ref[...] = z

    @pl.when(pl.program_id(0) == 0)
    def _():
        st_ref[...] = jnp.zeros_like(st_ref)

    ssum = jnp.sum(z, axis=0, keepdims=True)
    ssq = jnp.sum(z * z, axis=0, keepdims=True)
    st_ref[...] += jnp.concatenate(
        [ssum, ssq, jnp.zeros((6, D), jnp.float32)], axis=0)


def _mid_a(agg, tab, cnt, b):
    return pl.pallas_call(
        _mida_body,
        grid=(_NB,),
        in_specs=[
            pl.BlockSpec((NC, _BN, H), lambda i: (0, i, 0)),
            pl.BlockSpec((NC, _BN, H), lambda i: (0, i, 0)),
            pl.BlockSpec((_BN, 16), lambda i: (i, 0)),
            pl.BlockSpec((1, D), lambda i: (0, 0)),
        ],
        out_specs=[
            pl.BlockSpec((_BN, D), lambda i: (i, 0)),
            pl.BlockSpec((8, D), lambda i: (0, 0)),
        ],
        out_shape=[
            jax.ShapeDtypeStruct((N, D), jnp.float32),
            jax.ShapeDtypeStruct((8, D), jnp.float32),
        ],
    )(agg, tab, cnt, b)


def _bn_stats(st_ref):
    mean = st_ref[0:1] * (1.0 / N)
    ex2 = st_ref[1:2] * (1.0 / N)
    var = ex2 - mean * mean
    return mean, lax.rsqrt(var + 1e-5)


def _midb_body(z_ref, st_ref, cnt_ref, g_ref, be_ref, w_ref, o_ref):
    mean, rstd = _bn_stats(st_ref)
    y = (z_ref[...] - mean) * (rstd * g_ref[...]) + be_ref[...]
    y = jnp.maximum(y, 0.0)
    h = jnp.dot(y, w_ref[...], preferred_element_type=jnp.float32)
    hs = jnp.where(_rowmask(pl.program_id(0), _BNT),
                   h * _dinv_of(cnt_ref), 0.0)
    o_ref[0] = hs[:, :H]
    o_ref[1] = hs[:, H:]


def _mid_b(z, st, cnt, gamma, beta, W):
    return pl.pallas_call(
        _midb_body,
        grid=(_NBT,),
        in_specs=[
            pl.BlockSpec((_BNT, D), lambda i: (i, 0)),
            pl.BlockSpec((8, D), lambda i: (0, 0)),
            pl.BlockSpec((_BNT, 16), lambda i: (i, 0)),
            pl.BlockSpec((1, D), lambda i: (0, 0)),
            pl.BlockSpec((1, D), lambda i: (0, 0)),
            pl.BlockSpec((D, D), lambda i: (0, 0)),
        ],
        out_specs=pl.BlockSpec((NC, _BNT, H), lambda i: (0, i, 0)),
        out_shape=jax.ShapeDtypeStruct((NC, NP, H), jnp.float32),
    )(z, st, cnt, gamma, beta, W)


def _final_body(z_ref, st_ref, g_ref, be_ref, o_ref):
    mean, rstd = _bn_stats(st_ref)
    o_ref[...] = (z_ref[...] - mean) * (rstd * g_ref[...]) + be_ref[...]


def _final(z, st, gamma, beta):
    return pl.pallas_call(
        _final_body,
        grid=(_NB,),
        in_specs=[
            pl.BlockSpec((_BN, D), lambda i: (i, 0)),
            pl.BlockSpec((8, D), lambda i: (0, 0)),
            pl.BlockSpec((1, D), lambda i: (0, 0)),
            pl.BlockSpec((1, D), lambda i: (0, 0)),
        ],
        out_specs=pl.BlockSpec((_BN, D), lambda i: (i, 0)),
        out_shape=jax.ShapeDtypeStruct((N, D), jnp.float32),
    )(z, st, gamma, beta)


def _agg_ranges(tab, srcps, dstps, zeros_a):
    flat = tab.reshape(NC * NP, H)
    o0 = _agg_call(flat, srcps[0], dstps[0], zeros_a)
    o1 = _agg_call(flat, srcps[1], dstps[1], zeros_a)
    return jnp.concatenate([o0, o1], axis=1)


def kernel(x, edge_index, W1, b1, gamma1, beta1, W2, b2, gamma2, beta2):
    src = edge_index[0]
    dst = edge_index[1]

    pads = SCH * CH - ES
    pos = jnp.arange(SCH * CH, dtype=jnp.int32)

    dst_sub = jnp.concatenate(
        [dst.reshape(NS, ES),
         jnp.full((NS, pads), N, jnp.int32)], axis=1)
    src_sub = jnp.concatenate(
        [src.reshape(NS, ES), jnp.zeros((NS, pads), jnp.int32)], axis=1)

    padw = EWP - EW
    trash_w = N + (jnp.arange(padw, dtype=jnp.int32) % (NP - N))
    dstp_deg = jnp.concatenate(
        [dst.reshape(NC * NS, EW),
         jnp.broadcast_to(trash_w, (NC * NS, padw))],
        axis=1).reshape(NC, NS, EWP).astype(jnp.int32)

    zero_rows = N + (pos % (NP - N))
    srcps, dstps = [], []
    for r in range(NC):
        in_r = ((dst_sub >= r * NPA) &
                (dst_sub < min((r + 1) * NPA, N)))
        s_r = jnp.where(in_r, src_sub, zero_rows[None, :]).astype(jnp.int32)
        d_r = jnp.where(in_r, dst_sub - r * NPA,
                        pos[None, :] % NPA).astype(jnp.int32)
        srcps.append(jnp.stack([s_r, s_r + NP]))
        dstps.append(d_r.reshape(NS, SCH, CH))

    zeros_d = jnp.zeros((NP,), jnp.int32)
    zeros_a = jnp.zeros((ACC_ROWS, H), jnp.float32)

    cnt32 = _deg_call(dstp_deg, zeros_d)
    cnt = _dmerge(cnt32.reshape(NC * NS, NP))

    tab1 = _mm_scale(x, W1, cnt)
    agg1 = _agg_ranges(tab1, srcps, dstps, zeros_a)
    z1, st1 = _mid_a(agg1, tab1, cnt, b1.reshape(1, D))
    tab2 = _mid_b(z1, st1, cnt, gamma1.reshape(1, D), beta1.reshape(1, D),
                  W2)
    agg2 = _agg_ranges(tab2, srcps, dstps, zeros_a)
    z2, st2 = _mid_a(agg2, tab2, cnt, b2.reshape(1, D))
    return _final(z2, st2, gamma2.reshape(1, D), beta2.reshape(1, D))


# speedup vs baseline: 1.0079x; 1.0005x over previous
import jax
import jax.numpy as jnp
from jax import lax
from jax.experimental import pallas as pl
from jax.experimental.pallas import tpu as pltpu
from jax.experimental.pallas import tpu_sc as plsc

N = 10000
E = 160000
D = 256
H = D // 2
NC = 2
NS = 16
CH = 128
NP = 10112
NPA = 5120

ES = E // NS
SCH = (ES + CH - 1) // CH
ACC_ROWS = NPA // NS

_mesh = plsc.VectorSubcoreMesh(core_axis_name="c", subcore_axis_name="s")


EW = E // (NC * NS)
EWP = 5120


def _deg_body(dstp_hbm, zeros_hbm, out_hbm, dst_v, cnt_v):
    c = lax.axis_index("c")
    s = lax.axis_index("s")
    pltpu.sync_copy(zeros_hbm, cnt_v)
    pltpu.sync_copy(dstp_hbm.at[c, s], dst_v)

    def body(e, carry):
        idx = dst_v[pl.ds(e * 16, 16)]
        cnts, last = plsc.scan_count(idx)
        plsc.addupdate_scatter(cnt_v, [idx], cnts, mask=last)
        return carry

    lax.fori_loop(0, EWP // 16, body, 0)
    pltpu.sync_copy(cnt_v, out_hbm.at[c, s])


_deg_call = pl.kernel(
    _deg_body,
    out_type=jax.ShapeDtypeStruct((NC, NS, NP), jnp.int32),
    mesh=_mesh,
    scratch_types=[
        pltpu.VMEM((EWP,), jnp.int32),
        pltpu.VMEM((NP,), jnp.int32),
    ],
    compiler_params=pltpu.CompilerParams(needs_layout_passes=False),
)


def _dmerge_body(cnt32_ref, o_ref):
    deg = jnp.sum(cnt32_ref[...].astype(jnp.float32), axis=0) + 1.0
    dinv = lax.rsqrt(deg)
    o_ref[...] = jnp.broadcast_to(dinv[:, None], (128, 16))


def _dmerge(cnt32):
    return pl.pallas_call(
        _dmerge_body,
        grid=(NP // 128,),
        in_specs=[pl.BlockSpec((NC * NS, 128), lambda j: (0, j))],
        out_specs=pl.BlockSpec((128, 16), lambda j: (j, 0)),
        out_shape=jax.ShapeDtypeStruct((NP, 16), jnp.float32),
    )(cnt32)


def _agg_body(hs_hbm, srcp_hbm, dstp_hbm, zeros_hbm, out_hbm,
              src_v, dst_v, buf, acc, gsem, ssem):
    c = lax.axis_index("c")
    s = lax.axis_index("s")
    pltpu.sync_copy(zeros_hbm, acc.at[pl.ds(s * ACC_ROWS, ACC_ROWS)])
    pltpu.sync_copy(srcp_hbm.at[c, s], src_v)
    pltpu.sync_copy(dstp_hbm.at[s], dst_v)
    plsc.subcore_barrier()

    def gather(j):
        return pltpu.async_copy(
            hs_hbm.at[src_v.at[pl.ds(j * CH, CH)]], buf.at[j % 3], gsem)

    def scat(j):
        return pltpu.async_copy(
            buf.at[j % 3], acc.at[dst_v.at[j]], ssem, add=True)

    gd = {0: gather(0), 1: gather(1)}
    sd = None
    for j in range(SCH):
        if sd is not None:
            sd.wait()
        if j + 2 < SCH:
            gd[j + 2] = gather(j + 2)
        gd.pop(j).wait()
        sd = scat(j)
    sd.wait()
    plsc.subcore_barrier()
    pltpu.sync_copy(acc.at[pl.ds(s * ACC_ROWS, ACC_ROWS)],
                    out_hbm.at[c, pl.ds(s * ACC_ROWS, ACC_ROWS)])


_agg_call = pl.kernel(
    _agg_body,
    out_type=jax.ShapeDtypeStruct((NC, NPA, H), jnp.float32),
    mesh=_mesh,
    scratch_types=[
        pltpu.VMEM((SCH * CH,), jnp.int32),
        pltpu.VMEM((SCH, CH), jnp.int32),
        pltpu.VMEM((3, CH, H), jnp.float32),
        pltpu.VMEM_SHARED((NPA, H), jnp.float32),
        pltpu.SemaphoreType.DMA,
        pltpu.SemaphoreType.DMA,
    ],
)


_BN = 1000
_NB = N // _BN
_BNT = NP // 16
_NBT = 16


def _dinv_of(cnt_ref):
    return cnt_ref[:, 0:1]


def _rowmask(i, nrows):
    rows = i * nrows + lax.broadcasted_iota(jnp.int32, (nrows, 1), 0)
    return rows < N


def _mm_body(x_ref, w_ref, cnt_ref, o_ref):
    h = jnp.dot(x_ref[...], w_ref[...], preferred_element_type=jnp.float32)
    hs = jnp.where(_rowmask(pl.program_id(0), _BNT),
                   h * _dinv_of(cnt_ref), 0.0)
    o_ref[0] = hs[:, :H]
    o_ref[1] = hs[:, H:]


def _mm_scale(x, W, cnt):
    return pl.pallas_call(
        _mm_body,
        grid=(_NBT,),
        in_specs=[
            pl.BlockSpec((_BNT, D), lambda i: (i, 0)),
            pl.BlockSpec((D, D), lambda i: (0, 0)),
            pl.BlockSpec((_BNT, 16), lambda i: (i, 0)),
        ],
        out_specs=pl.BlockSpec((NC, _BNT, H), lambda i: (0, i, 0)),
        out_shape=jax.ShapeDtypeStruct((NC, NP, H), jnp.float32),
    )(x, W, cnt)


def _mida_body(agg_ref, tab_ref, cnt_ref, b_ref, z_ref, st_ref):
    dinv = _dinv_of(cnt_ref)
    a = jnp.concatenate([agg_ref[0], agg_ref[1]], axis=1)
    hs = jnp.concatenate([tab_ref[0], tab_ref[1]], axis=1)
    z = (a + hs) * dinv + b_ref[...]
    z_ref[...] = z

    @pl.when(pl.program_id(0) == 0)
    def _():
        st_ref[...] = jnp.zeros_like(st_ref)

    ssum = jnp.sum(z, axis=0, keepdims=True)
    ssq = jnp.sum(z * z, axis=0, keepdims=True)
    st_ref[...] += jnp.concatenate(
        [ssum, ssq, jnp.zeros((6, D), jnp.float32)], axis=0)


def _mid_a(agg, tab, cnt, b):
    return pl.pallas_call(
        _mida_body,
        grid=(_NB,),
        in_specs=[
            pl.BlockSpec((NC, _BN, H), lambda i: (0, i, 0)),
            pl.BlockSpec((NC, _BN, H), lambda i: (0, i, 0)),
            pl.BlockSpec((_BN, 16), lambda i: (i, 0)),
            pl.BlockSpec((1, D), lambda i: (0, 0)),
        ],
        out_specs=[
            pl.BlockSpec((_BN, D), lambda i: (i, 0)),
            pl.BlockSpec((8, D), lambda i: (0, 0)),
        ],
        out_shape=[
            jax.ShapeDtypeStruct((N, D), jnp.float32),
            jax.ShapeDtypeStruct((8, D), jnp.float32),
        ],
    )(agg, tab, cnt, b)


def _bn_stats(st_ref):
    mean = st_ref[0:1] * (1.0 / N)
    ex2 = st_ref[1:2] * (1.0 / N)
    var = ex2 - mean * mean
    return mean, lax.rsqrt(var + 1e-5)


def _midb_body(z_ref, st_ref, cnt_ref, g_ref, be_ref, w_ref, o_ref):
    mean, rstd = _bn_stats(st_ref)
    y = (z_ref[...] - mean) * (rstd * g_ref[...]) + be_ref[...]
    y = jnp.maximum(y, 0.0)
    h = jnp.dot(y, w_ref[...], preferred_element_type=jnp.float32)
    hs = jnp.where(_rowmask(pl.program_id(0), _BNT),
                   h * _dinv_of(cnt_ref), 0.0)
    o_ref[0] = hs[:, :H]
    o_ref[1] = hs[:, H:]


def _mid_b(z, st, cnt, gamma, beta, W):
    return pl.pallas_call(
        _midb_body,
        grid=(_NBT,),
        in_specs=[
            pl.BlockSpec((_BNT, D), lambda i: (i, 0)),
            pl.BlockSpec((8, D), lambda i: (0, 0)),
            pl.BlockSpec((_BNT, 16), lambda i: (i, 0)),
            pl.BlockSpec((1, D), lambda i: (0, 0)),
            pl.BlockSpec((1, D), lambda i: (0, 0)),
            pl.BlockSpec((D, D), lambda i: (0, 0)),
        ],
        out_specs=pl.BlockSpec((NC, _BNT, H), lambda i: (0, i, 0)),
        out_shape=jax.ShapeDtypeStruct((NC, NP, H), jnp.float32),
    )(z, st, cnt, gamma, beta, W)


def _final_body(z_ref, st_ref, g_ref, be_ref, o_ref):
    mean, rstd = _bn_stats(st_ref)
    o_ref[...] = (z_ref[...] - mean) * (rstd * g_ref[...]) + be_ref[...]


def _final(z, st, gamma, beta):
    return pl.pallas_call(
        _final_body,
        grid=(_NB,),
        in_specs=[
            pl.BlockSpec((_BN, D), lambda i: (i, 0)),
            pl.BlockSpec((8, D), lambda i: (0, 0)),
            pl.BlockSpec((1, D), lambda i: (0, 0)),
            pl.BlockSpec((1, D), lambda i: (0, 0)),
        ],
        out_specs=pl.BlockSpec((_BN, D), lambda i: (i, 0)),
        out_shape=jax.ShapeDtypeStruct((N, D), jnp.float32),
    )(z, st, gamma, beta)


def _agg_ranges(tab, srcps, dstps, zeros_a):
    flat = tab.reshape(NC * NP, H)
    o0 = _agg_call(flat, srcps[0], dstps[0], zeros_a)
    o1 = _agg_call(flat, srcps[1], dstps[1], zeros_a)
    return jnp.concatenate([o0, o1], axis=1)


def kernel(x, edge_index, W1, b1, gamma1, beta1, W2, b2, gamma2, beta2):
    src = edge_index[0]
    dst = edge_index[1]

    pads = SCH * CH - ES
    pos = jnp.arange(SCH * CH, dtype=jnp.int32)

    dst_sub = jnp.concatenate(
        [dst.reshape(NS, ES),
         jnp.full((NS, pads), N, jnp.int32)], axis=1)
    src_sub = jnp.concatenate(
        [src.reshape(NS, ES), jnp.zeros((NS, pads), jnp.int32)], axis=1)

    padw = EWP - EW
    trash_w = N + (jnp.arange(padw, dtype=jnp.int32) % (NP - N))
    dstp_deg = jnp.concatenate(
        [dst.reshape(NC * NS, EW),
         jnp.broadcast_to(trash_w, (NC * NS, padw))],
        axis=1).reshape(NC, NS, EWP).astype(jnp.int32)

    zero_rows = N + (pos % (NP - N))
    srcps, dstps = [], []
    for r in range(NC):
        in_r = ((dst_sub >= r * NPA) &
                (dst_sub < min((r + 1) * NPA, N)))
        s_r = jnp.where(in_r, src_sub, zero_rows[None, :]).astype(jnp.int32)
        d_r = jnp.where(in_r, dst_sub - r * NPA,
                        pos[None, :] % NPA).astype(jnp.int32)
        srcps.append(jnp.stack([s_r, s_r + NP]))
        dstps.append(d_r.reshape(NS, SCH, CH))

    zeros_d = jnp.zeros((NP,), jnp.int32)
    zeros_a = jnp.zeros((ACC_ROWS, H), jnp.float32)

    cnt32 = _deg_call(dstp_deg, zeros_d)
    cnt = _dmerge(cnt32.reshape(NC * NS, NP))

    tab1 = _mm_scale(x, W1, cnt)
    agg1 = _agg_ranges(tab1, srcps, dstps, zeros_a)
    z1, st1 = _mid_a(agg1, tab1, cnt, b1.reshape(1, D))
    tab2 = _mid_b(z1, st1, cnt, gamma1.reshape(1, D), beta1.reshape(1, D),
                  W2)
    agg2 = _agg_ranges(tab2, srcps, dstps, zeros_a)
    z2, st2 = _mid_a(agg2, tab2, cnt, b2.reshape(1, D))
    return _final(z2, st2, gamma2.reshape(1, D), beta2.reshape(1, D))
